# Initial kernel scaffold; baseline (speedup 1.0000x reference)
import functools

import jax
import jax.numpy as jnp
from jax import lax
from jax.experimental import pallas as pl
from jax.experimental.pallas import tpu as pltpu
from jax.experimental.pallas import tpu_sc as plsc

_N = 10000
_C = 128
_G = 64
_E = 320000

_NSC = 2
_NTEC = 16
_NW = _NSC * _NTEC
_EPC = 128

_NPAD = 10016
_RPT = _NPAD // _NTEC

_EPAD = ((_E + _NW * _EPC - 1) // (_NW * _EPC)) * (_NW * _EPC)
_NPT = _EPAD // _NW
_NCH = _NPT // _EPC

_BLK = 2000


def _mm_body(h_ref, w_ref, o_ref):
    o_ref[...] = jnp.dot(h_ref[...], w_ref[...],
                         preferred_element_type=jnp.float32)


def _mm(h, w):
    grid = _N // _BLK
    return pl.pallas_call(
        _mm_body,
        grid=(grid,),
        in_specs=[
            pl.BlockSpec((_BLK, _C), lambda i: (i, 0)),
            pl.BlockSpec((_C, _C), lambda i: (0, 0)),
        ],
        out_specs=pl.BlockSpec((_BLK, _C), lambda i: (i, 0)),
        out_shape=jax.ShapeDtypeStruct((_N, _C), jnp.float32),
    )(h, w)


def _scatter_body(m_hbm, src_hbm, dst_hbm, out_hbm, agg_sh, src_v, dst_v,
                  rows_v, sem):
    cid = lax.axis_index("c")
    sid = lax.axis_index("s")
    wid = cid * _NTEC + sid

    def _zrow(i, _):
        def _zcol(j, _):
            rows_v[i, pl.ds(j * 16, 16)] = jnp.zeros((16,), jnp.float32)
            return 0
        return lax.fori_loop(0, _C // 16, _zcol, 0)

    lax.fori_loop(0, _EPC, _zrow, 0)

    row0 = sid * _RPT
    for k in range(_RPT // _EPC):
        pltpu.sync_copy(rows_v, agg_sh.at[pl.ds(row0 + k * _EPC, _EPC)])
    rem = _RPT % _EPC
    if rem:
        pltpu.sync_copy(rows_v.at[pl.ds(0, rem)],
                        agg_sh.at[pl.ds(row0 + (_RPT // _EPC) * _EPC, rem)])

    plsc.subcore_barrier()

    base = wid * _NPT

    def _chunk(j, _):
        off = base + j * _EPC
        pltpu.sync_copy(src_hbm.at[pl.ds(off, _EPC)], src_v)
        pltpu.sync_copy(dst_hbm.at[pl.ds(off, _EPC)], dst_v)
        pltpu.async_copy(m_hbm.at[src_v], rows_v, sem).wait()
        pltpu.sync_copy(rows_v, agg_sh.at[dst_v], add=True)
        return 0

    lax.fori_loop(0, _NCH, _chunk, 0)

    plsc.subcore_barrier()

    pltpu.sync_copy(agg_sh.at[pl.ds(row0, _RPT)],
                    out_hbm.at[cid].at[pl.ds(row0, _RPT)])


_sc_scatter = functools.partial(
    pl.kernel,
    out_type=jax.ShapeDtypeStruct((_NSC, _NPAD, _C), jnp.float32),
    mesh=plsc.VectorSubcoreMesh(core_axis_name="c", subcore_axis_name="s",
                                num_cores=_NSC, num_subcores=_NTEC),
    scratch_types=[
        pltpu.VMEM_SHARED((_NPAD, _C), jnp.float32),
        pltpu.VMEM((_EPC,), jnp.int32),
        pltpu.VMEM((_EPC,), jnp.int32),
        pltpu.VMEM((_EPC, _C), jnp.float32),
        pltpu.SemaphoreType.DMA,
    ],
)(_scatter_body)


def _gru_body(agg_ref, h_ref, wih_ref, whh_ref, bih_ref, bhh_ref, o_ref):
    a = agg_ref[0] + agg_ref[1]
    h = h_ref[...]
    gi = jnp.dot(a, wih_ref[...],
                 preferred_element_type=jnp.float32) + bih_ref[...]
    gh = jnp.dot(h, whh_ref[...],
                 preferred_element_type=jnp.float32) + bhh_ref[...]
    r = jax.nn.sigmoid(gi[:, :_C] + gh[:, :_C])
    z = jax.nn.sigmoid(gi[:, _C:2 * _C] + gh[:, _C:2 * _C])
    nn_ = jnp.tanh(gi[:, 2 * _C:] + r * gh[:, 2 * _C:])
    o_ref[...] = (1.0 - z) * nn_ + z * h


def _gru(agg2, h, wihT, whhT, bih, bhh):
    grid = _N // _BLK
    return pl.pallas_call(
        _gru_body,
        grid=(grid,),
        in_specs=[
            pl.BlockSpec((_NSC, _BLK, _C), lambda i: (0, i, 0)),
            pl.BlockSpec((_BLK, _C), lambda i: (i, 0)),
            pl.BlockSpec((_C, 3 * _C), lambda i: (0, 0)),
            pl.BlockSpec((_C, 3 * _C), lambda i: (0, 0)),
            pl.BlockSpec((1, 3 * _C), lambda i: (0, 0)),
            pl.BlockSpec((1, 3 * _C), lambda i: (0, 0)),
        ],
        out_specs=pl.BlockSpec((_BLK, _C), lambda i: (i, 0)),
        out_shape=jax.ShapeDtypeStruct((_N, _C), jnp.float32),
    )(agg2, h, wihT, whhT, bih, bhh)


def _pool_body(h_ref, b_ref, wg_ref, bg_ref, wl_ref, bl_ref, o_ref):
    h = h_ref[...]
    bvec = b_ref[...]
    iota_g = lax.broadcasted_iota(jnp.int32, (_N, _G), 1)
    msk = bvec == iota_g
    p = msk.astype(jnp.float32)
    gate = jnp.sum(h * wg_ref[...], axis=1, keepdims=True) + bg_ref[...]
    gm = jnp.max(jnp.where(msk, gate, -1e30), axis=0, keepdims=True)
    gmn = jnp.sum(p * gm, axis=1, keepdims=True)
    ex = jnp.exp(gate - gmn)
    den = jnp.sum(p * ex, axis=0, keepdims=True)
    denn = jnp.sum(p * den, axis=1, keepdims=True)
    alpha = ex / denn
    out1 = lax.dot_general(p, alpha * h, (((0,), (0,)), ((), ())),
                           preferred_element_type=jnp.float32)
    x2 = jnp.tanh(jnp.dot(h, wl_ref[...],
                          preferred_element_type=jnp.float32) + bl_ref[...])
    out2 = lax.dot_general(p, x2, (((0,), (0,)), ((), ())),
                           preferred_element_type=jnp.float32)
    o_ref[...] = jnp.tanh(out1 * out2)


def _pool(h, batch2, wg, bg, wlT, bl):
    return pl.pallas_call(
        _pool_body,
        in_specs=[
            pl.BlockSpec((_N, _C), lambda: (0, 0)),
            pl.BlockSpec((_N, 1), lambda: (0, 0)),
            pl.BlockSpec((1, _C), lambda: (0, 0)),
            pl.BlockSpec((1, 1), lambda: (0, 0)),
            pl.BlockSpec((_C, _C), lambda: (0, 0)),
            pl.BlockSpec((1, _C), lambda: (0, 0)),
        ],
        out_specs=pl.BlockSpec((_G, _C), lambda: (0, 0)),
        out_shape=jax.ShapeDtypeStruct((_G, _C), jnp.float32),
    )(h, batch2, wg, bg, wlT, bl)


def kernel(x, edge_index, batch, W, W_ih, W_hh, b_ih, b_hh, Wg, bg, Wl, bl):
    src, dst = edge_index[0], edge_index[1]
    pad = _EPAD - _E
    srcp = jnp.concatenate([src, jnp.zeros((pad,), jnp.int32)])
    dstp = jnp.concatenate([dst, jnp.full((pad,), _N, jnp.int32)])

    wihT = W_ih.T
    whhT = W_hh.T
    bih = b_ih.reshape(1, 3 * _C)
    bhh = b_hh.reshape(1, 3 * _C)
    wg = Wg.reshape(1, _C)
    bg2 = bg.reshape(1, 1)
    wlT = Wl.T
    bl2 = bl.reshape(1, _C)
    batch2 = batch.reshape(_N, 1)

    h = x
    for i in range(W.shape[0]):
        m = _mm(h, W[i])
        agg2 = _sc_scatter(m, srcp, dstp)
        h = _gru(agg2, h, wihT, whhT, bih, bhh)

    return _pool(h, batch2, wg, bg2, wlT, bl2)


# speedup vs baseline: 3.7128x; 3.7128x over previous
import functools

import jax
import jax.numpy as jnp
from jax import lax
from jax.experimental import pallas as pl
from jax.experimental.pallas import tpu as pltpu
from jax.experimental.pallas import tpu_sc as plsc

_N = 10000
_C = 128
_G = 64
_E = 320000

_NSC = 2
_NTEC = 16
_NW = _NSC * _NTEC
_EPC = 128

_NPAD = 10112
_RPT = _NPAD // _NTEC

_EPAD = ((_E + _NW * _EPC - 1) // (_NW * _EPC)) * (_NW * _EPC)
_NPT = _EPAD // _NW
_NCH = _NPT // _EPC

_BLK = 2000


def _mm_body(h_ref, w_ref, o_ref):
    o_ref[...] = jnp.dot(h_ref[...], w_ref[...],
                         preferred_element_type=jnp.float32)


def _mm(h, w):
    grid = _N // _BLK
    return pl.pallas_call(
        _mm_body,
        grid=(grid,),
        in_specs=[
            pl.BlockSpec((_BLK, _C), lambda i: (i, 0)),
            pl.BlockSpec((_C, _C), lambda i: (0, 0)),
        ],
        out_specs=pl.BlockSpec((_BLK, _C), lambda i: (i, 0)),
        out_shape=jax.ShapeDtypeStruct((_N, _C), jnp.float32),
    )(h, w)


def _scatter_body(m_hbm, src_hbm, dst_hbm, out_hbm, agg_sh, src_v, dst_v,
                  rows_v, sem):
    cid = lax.axis_index("c")
    sid = lax.axis_index("s")
    wid = cid * _NTEC + sid

    def _zrow(i, _):
        def _zcol(j, _):
            rows_v[i, pl.ds(j * 16, 16)] = jnp.zeros((16,), jnp.float32)
            return 0
        return lax.fori_loop(0, _C // 16, _zcol, 0)

    lax.fori_loop(0, _EPC, _zrow, 0)

    row0 = sid * _RPT
    for k in range(_RPT // _EPC):
        pltpu.sync_copy(rows_v, agg_sh.at[pl.ds(row0 + k * _EPC, _EPC)])
    rem = _RPT % _EPC
    if rem:
        pltpu.sync_copy(rows_v.at[pl.ds(0, rem)],
                        agg_sh.at[pl.ds(row0 + (_RPT // _EPC) * _EPC, rem)])

    plsc.subcore_barrier()

    base = wid * _NPT

    def _chunk(j, _):
        off = base + j * _EPC
        pltpu.sync_copy(src_hbm.at[pl.ds(off, _EPC)], src_v)
        pltpu.sync_copy(dst_hbm.at[pl.ds(off, _EPC)], dst_v)
        pltpu.async_copy(m_hbm.at[src_v], rows_v, sem).wait()
        pltpu.sync_copy(rows_v, agg_sh.at[dst_v], add=True)
        return 0

    lax.fori_loop(0, _NCH, _chunk, 0)

    plsc.subcore_barrier()

    pltpu.sync_copy(agg_sh.at[pl.ds(row0, _RPT)],
                    out_hbm.at[cid].at[pl.ds(row0, _RPT)])


_sc_scatter = functools.partial(
    pl.kernel,
    out_type=jax.ShapeDtypeStruct((_NSC, _NPAD, _C), jnp.float32),
    mesh=plsc.VectorSubcoreMesh(core_axis_name="c", subcore_axis_name="s",
                                num_cores=_NSC, num_subcores=_NTEC),
    scratch_types=[
        pltpu.VMEM_SHARED((_NPAD, _C), jnp.float32),
        pltpu.VMEM((_EPC,), jnp.int32),
        pltpu.VMEM((_EPC,), jnp.int32),
        pltpu.VMEM((_EPC, _C), jnp.float32),
        pltpu.SemaphoreType.DMA,
    ],
)(_scatter_body)


def _gru_body(agg_ref, h_ref, wih_ref, whh_ref, bih_ref, bhh_ref, o_ref):
    a = agg_ref[0] + agg_ref[1]
    h = h_ref[...]
    gi = jnp.dot(a, wih_ref[...],
                 preferred_element_type=jnp.float32) + bih_ref[...]
    gh = jnp.dot(h, whh_ref[...],
                 preferred_element_type=jnp.float32) + bhh_ref[...]
    r = jax.nn.sigmoid(gi[:, :_C] + gh[:, :_C])
    z = jax.nn.sigmoid(gi[:, _C:2 * _C] + gh[:, _C:2 * _C])
    nn_ = jnp.tanh(gi[:, 2 * _C:] + r * gh[:, 2 * _C:])
    o_ref[...] = (1.0 - z) * nn_ + z * h


def _gru(agg2, h, wihT, whhT, bih, bhh):
    grid = _N // _BLK
    return pl.pallas_call(
        _gru_body,
        grid=(grid,),
        in_specs=[
            pl.BlockSpec((_NSC, _BLK, _C), lambda i: (0, i, 0)),
            pl.BlockSpec((_BLK, _C), lambda i: (i, 0)),
            pl.BlockSpec((_C, 3 * _C), lambda i: (0, 0)),
            pl.BlockSpec((_C, 3 * _C), lambda i: (0, 0)),
            pl.BlockSpec((1, 3 * _C), lambda i: (0, 0)),
            pl.BlockSpec((1, 3 * _C), lambda i: (0, 0)),
        ],
        out_specs=pl.BlockSpec((_BLK, _C), lambda i: (i, 0)),
        out_shape=jax.ShapeDtypeStruct((_N, _C), jnp.float32),
    )(agg2, h, wihT, whhT, bih, bhh)


def _pool_body(h_ref, b_ref, wg_ref, bg_ref, wl_ref, bl_ref, o_ref):
    h = h_ref[...]
    bvec = b_ref[...]
    iota_g = lax.broadcasted_iota(jnp.int32, (_N, _G), 1)
    msk = bvec == iota_g
    p = msk.astype(jnp.float32)
    gate = jnp.sum(h * wg_ref[...], axis=1, keepdims=True) + bg_ref[...]
    gm = jnp.max(jnp.where(msk, gate, -1e30), axis=0, keepdims=True)
    gmn = jnp.sum(p * gm, axis=1, keepdims=True)
    ex = jnp.exp(gate - gmn)
    den = jnp.sum(p * ex, axis=0, keepdims=True)
    denn = jnp.sum(p * den, axis=1, keepdims=True)
    alpha = ex / denn
    out1 = lax.dot_general(p, alpha * h, (((0,), (0,)), ((), ())),
                           preferred_element_type=jnp.float32)
    x2 = jnp.tanh(jnp.dot(h, wl_ref[...],
                          preferred_element_type=jnp.float32) + bl_ref[...])
    out2 = lax.dot_general(p, x2, (((0,), (0,)), ((), ())),
                           preferred_element_type=jnp.float32)
    o_ref[...] = jnp.tanh(out1 * out2)


def _pool(h, batch2, wg, bg, wlT, bl):
    return pl.pallas_call(
        _pool_body,
        in_specs=[
            pl.BlockSpec((_N, _C), lambda: (0, 0)),
            pl.BlockSpec((_N, 1), lambda: (0, 0)),
            pl.BlockSpec((1, _C), lambda: (0, 0)),
            pl.BlockSpec((1, 1), lambda: (0, 0)),
            pl.BlockSpec((_C, _C), lambda: (0, 0)),
            pl.BlockSpec((1, _C), lambda: (0, 0)),
        ],
        out_specs=pl.BlockSpec((_G, _C), lambda: (0, 0)),
        out_shape=jax.ShapeDtypeStruct((_G, _C), jnp.float32),
    )(h, batch2, wg, bg, wlT, bl)


def kernel(x, edge_index, batch, W, W_ih, W_hh, b_ih, b_hh, Wg, bg, Wl, bl):
    src, dst = edge_index[0], edge_index[1]
    pad = _EPAD - _E
    srcp = jnp.concatenate([src, jnp.zeros((pad,), jnp.int32)])
    dstp = jnp.concatenate([dst, jnp.full((pad,), _N, jnp.int32)])

    wihT = W_ih.T
    whhT = W_hh.T
    bih = b_ih.reshape(1, 3 * _C)
    bhh = b_hh.reshape(1, 3 * _C)
    wg = Wg.reshape(1, _C)
    bg2 = bg.reshape(1, 1)
    wlT = Wl.T
    bl2 = bl.reshape(1, _C)
    batch2 = batch.reshape(_N, 1)

    h = x
    for i in range(W.shape[0]):
        m = _mm(h, W[i])
        agg2 = _sc_scatter(m, srcp, dstp)
        h = _gru(agg2, h, wihT, whhT, bih, bhh)

    return _pool(h, batch2, wg, bg2, wlT, bl2)
